# Initial kernel scaffold; baseline (speedup 1.0000x reference)
import jax
import jax.numpy as jnp
from jax.experimental import pallas as pl


def kernel(x, edge_index, W1, b1, W2, b2, W3, b3):
    raise NotImplementedError("write your pallas kernel here")


# speedup vs baseline: 9.1331x; 9.1331x over previous
import functools
import jax
import jax.numpy as jnp
from jax import lax
from jax.experimental import pallas as pl
from jax.experimental.pallas import tpu as pltpu
from jax.experimental.pallas import tpu_sc as plsc

N = 10000
F = 128
E = 320000
NC = 2
NS = 16
CHUNK = 128
CHUNKS_PER_TILE = -(-E // (NC * NS * CHUNK))
EPT = CHUNKS_PER_TILE * CHUNK
E_PAD = NC * NS * EPT
ROWS_PAD = 10240
ZROWS = ROWS_PAD // NS

_sc_mesh = plsc.VectorSubcoreMesh(core_axis_name="c", subcore_axis_name="s")


@jax.jit
def _deg_sc(dst, zeros16):
    @functools.partial(
        pl.kernel,
        out_type=jax.ShapeDtypeStruct((NC, ROWS_PAD, 16), jnp.float32),
        mesh=_sc_mesh,
        scratch_types=[
            pltpu.VMEM((CHUNK,), jnp.int32),
            pltpu.VMEM((CHUNK, 16), jnp.float32),
            pltpu.VMEM_SHARED((ROWS_PAD, 16), jnp.float32),
            pltpu.SemaphoreType.DMA,
        ],
    )
    def deg_kernel(dst_hbm, z_hbm, out_hbm, didx_v, ones_v, acc, sem):
        c = lax.axis_index("c")
        s = lax.axis_index("s")

        @pl.loop(0, CHUNK)
        def _(i):
            ones_v[i, :] = jnp.full((16,), 1.0, jnp.float32)

        pltpu.sync_copy(z_hbm.at[pl.ds(s * ZROWS, ZROWS)],
                        acc.at[pl.ds(s * ZROWS, ZROWS)])
        plsc.subcore_barrier()

        base = (c * NS + s) * EPT

        @pl.loop(0, CHUNKS_PER_TILE)
        def _(j):
            pltpu.sync_copy(dst_hbm.at[pl.ds(base + j * CHUNK, CHUNK)], didx_v)
            pltpu.sync_copy(ones_v, acc.at[didx_v], add=True)

        plsc.subcore_barrier()
        pltpu.sync_copy(acc.at[pl.ds(s * ZROWS, ZROWS)],
                        out_hbm.at[c].at[pl.ds(s * ZROWS, ZROWS)])

    return deg_kernel(dst, zeros16)


@jax.jit
def _agg_sc(u, src, dst, zeros128):
    @functools.partial(
        pl.kernel,
        out_type=jax.ShapeDtypeStruct((NC, ROWS_PAD, F), jnp.float32),
        mesh=_sc_mesh,
        scratch_types=[
            pltpu.VMEM((CHUNK,), jnp.int32),
            pltpu.VMEM((CHUNK,), jnp.int32),
            pltpu.VMEM((CHUNK, F), jnp.float32),
            pltpu.VMEM_SHARED((ROWS_PAD, F), jnp.float32),
            pltpu.SemaphoreType.DMA,
        ],
    )
    def agg_kernel(u_hbm, src_hbm, dst_hbm, z_hbm, out_hbm,
                   sidx_v, didx_v, rows_v, acc, sem):
        c = lax.axis_index("c")
        s = lax.axis_index("s")

        pltpu.sync_copy(z_hbm.at[pl.ds(s * ZROWS, ZROWS)],
                        acc.at[pl.ds(s * ZROWS, ZROWS)])
        plsc.subcore_barrier()

        base = (c * NS + s) * EPT

        @pl.loop(0, CHUNKS_PER_TILE)
        def _(j):
            off = base + j * CHUNK
            pltpu.sync_copy(src_hbm.at[pl.ds(off, CHUNK)], sidx_v)
            pltpu.sync_copy(dst_hbm.at[pl.ds(off, CHUNK)], didx_v)
            pltpu.async_copy(u_hbm.at[sidx_v], rows_v, sem).wait()
            pltpu.sync_copy(rows_v, acc.at[didx_v], add=True)

        plsc.subcore_barrier()
        pltpu.sync_copy(acc.at[pl.ds(s * ZROWS, ZROWS)],
                        out_hbm.at[c].at[pl.ds(s * ZROWS, ZROWS)])

    return agg_kernel(u, src, dst, zeros128)


R = 1000


def _dinv_block(deg_ref):
    d = deg_ref[0, :, 0:1] + deg_ref[1, :, 0:1] + 1.0
    return lax.rsqrt(d)


def _k1_body(x_ref, w_ref, deg_ref, u_ref):
    dinv = _dinv_block(deg_ref)
    h = jnp.dot(x_ref[...], w_ref[...], preferred_element_type=jnp.float32,
                precision=lax.Precision.HIGHEST)
    u_ref[...] = h * dinv


@jax.jit
def _k1(x, w1, deg2):
    return pl.pallas_call(
        _k1_body,
        grid=(N // R,),
        in_specs=[
            pl.BlockSpec((R, F), lambda i: (i, 0)),
            pl.BlockSpec((F, F), lambda i: (0, 0)),
            pl.BlockSpec((NC, R, 16), lambda i: (0, i, 0)),
        ],
        out_specs=pl.BlockSpec((R, F), lambda i: (i, 0)),
        out_shape=jax.ShapeDtypeStruct((N, F), jnp.float32),
    )(x, w1, deg2)


def _k23_body(deg_ref, acc_ref, u_ref, b_ref, w_ref, un_ref):
    dinv = _dinv_block(deg_ref)
    sagg = acc_ref[0] + acc_ref[1] + u_ref[...]
    y = jnp.maximum(sagg * dinv + b_ref[...], 0.0)
    h = jnp.dot(y, w_ref[...], preferred_element_type=jnp.float32,
                precision=lax.Precision.HIGHEST)
    un_ref[...] = h * dinv


@jax.jit
def _k23(deg2, accp, u, b_prev, w_next):
    return pl.pallas_call(
        _k23_body,
        grid=(N // R,),
        in_specs=[
            pl.BlockSpec((NC, R, 16), lambda i: (0, i, 0)),
            pl.BlockSpec((NC, R, F), lambda i: (0, i, 0)),
            pl.BlockSpec((R, F), lambda i: (i, 0)),
            pl.BlockSpec((1, F), lambda i: (0, 0)),
            pl.BlockSpec((F, F), lambda i: (0, 0)),
        ],
        out_specs=pl.BlockSpec((R, F), lambda i: (i, 0)),
        out_shape=jax.ShapeDtypeStruct((N, F), jnp.float32),
    )(deg2, accp, u, b_prev, w_next)


def _k4_body(deg_ref, acc_ref, u_ref, b_ref, out_ref):
    dinv = _dinv_block(deg_ref)
    sagg = acc_ref[0] + acc_ref[1] + u_ref[...]
    out_ref[...] = sagg * dinv + b_ref[...]


@jax.jit
def _k4(deg2, accp, u, b_last):
    return pl.pallas_call(
        _k4_body,
        grid=(N // R,),
        in_specs=[
            pl.BlockSpec((NC, R, 16), lambda i: (0, i, 0)),
            pl.BlockSpec((NC, R, F), lambda i: (0, i, 0)),
            pl.BlockSpec((R, F), lambda i: (i, 0)),
            pl.BlockSpec((1, F), lambda i: (0, 0)),
        ],
        out_specs=pl.BlockSpec((R, F), lambda i: (i, 0)),
        out_shape=jax.ShapeDtypeStruct((N, F), jnp.float32),
    )(deg2, accp, u, b_last)


def kernel(x, edge_index, W1, b1, W2, b2, W3, b3):
    src = edge_index[0].astype(jnp.int32)
    dst = edge_index[1].astype(jnp.int32)
    pad = E_PAD - E
    src = jnp.concatenate([src, jnp.zeros((pad,), jnp.int32)])
    dst = jnp.concatenate([dst, jnp.full((pad,), N, jnp.int32)])

    zeros16 = jnp.zeros((ROWS_PAD, 16), jnp.float32)
    zeros128 = jnp.zeros((ROWS_PAD, F), jnp.float32)

    deg2 = _deg_sc(dst, zeros16)

    u = _k1(x, W1, deg2)
    accp = _agg_sc(u, src, dst, zeros128)
    u = _k23(deg2, accp, u, b1.reshape(1, F), W2)
    accp = _agg_sc(u, src, dst, zeros128)
    u = _k23(deg2, accp, u, b2.reshape(1, F), W3)
    accp = _agg_sc(u, src, dst, zeros128)
    return _k4(deg2, accp, u, b3.reshape(1, F))


# speedup vs baseline: 16.4085x; 1.7966x over previous
import functools
import jax
import jax.numpy as jnp
from jax import lax
from jax.experimental import pallas as pl
from jax.experimental.pallas import tpu as pltpu
from jax.experimental.pallas import tpu_sc as plsc

N = 10000
F = 128
E = 320000
NC = 2
NS = 16
CHUNK = 128
DEPTH = 2
IBLK = 16
NCHUNKS = NC * NS * IBLK * (-(-E // (CHUNK * NC * NS * IBLK)))
E_PAD = NCHUNKS * CHUNK
CPT = NCHUNKS // (NC * NS)
ROWS_PAD = 10240
ZROWS = ROWS_PAD // NS

_sc_mesh = plsc.VectorSubcoreMesh(core_axis_name="c", subcore_axis_name="s")


@jax.jit
def _deg_sc(dst, zeros16):
    @functools.partial(
        pl.kernel,
        out_type=jax.ShapeDtypeStruct((NC, ROWS_PAD, 16), jnp.float32),
        mesh=_sc_mesh,
        scratch_types=[
            pltpu.VMEM_SHARED((ROWS_PAD, 16), jnp.float32),
            pltpu.VMEM((CPT, CHUNK), jnp.int32),
            pltpu.VMEM((CHUNK,), jnp.int32),
            pltpu.VMEM((CHUNK, 16), jnp.float32),
            pltpu.SemaphoreType.DMA,
        ],
    )
    def deg_kernel(dst_hbm, z_hbm, out_hbm, acc, didx_b, didx_v, ones_v, sem):
        c = lax.axis_index("c")
        s = lax.axis_index("s")

        @pl.loop(0, CHUNK)
        def _(i):
            ones_v[i, :] = jnp.full((16,), 1.0, jnp.float32)

        cbase = (c * NS + s) * CPT
        pltpu.sync_copy(dst_hbm.at[pl.ds(cbase, CPT)], didx_b)

        pltpu.sync_copy(z_hbm.at[pl.ds(s * ZROWS, ZROWS)],
                        acc.at[pl.ds(s * ZROWS, ZROWS)])
        plsc.subcore_barrier()

        @pl.loop(0, CPT)
        def _(j):
            for i in range(CHUNK // 16):
                didx_v[pl.ds(i * 16, 16)] = didx_b[j, pl.ds(i * 16, 16)]
            pltpu.sync_copy(ones_v, acc.at[didx_v], add=True)

        plsc.subcore_barrier()
        pltpu.sync_copy(acc.at[pl.ds(s * ZROWS, ZROWS)],
                        out_hbm.at[c].at[pl.ds(s * ZROWS, ZROWS)])

    return deg_kernel(dst, zeros16)


@jax.jit
def _agg_sc(u, src, dst, zeros128):
    @functools.partial(
        pl.kernel,
        out_type=jax.ShapeDtypeStruct((NC, ROWS_PAD, F), jnp.float32),
        mesh=_sc_mesh,
        scratch_types=[
            pltpu.VMEM_SHARED((ROWS_PAD, F), jnp.float32),
            pltpu.VMEM((IBLK, CHUNK), jnp.int32),
            pltpu.VMEM((IBLK, CHUNK), jnp.int32),
            pltpu.VMEM((CHUNK,), jnp.int32),
            pltpu.VMEM((CHUNK,), jnp.int32),
            pltpu.VMEM((CHUNK, F), jnp.float32),
            pltpu.VMEM((CHUNK, F), jnp.float32),
            pltpu.SemaphoreType.DMA,
            pltpu.SemaphoreType.DMA,
        ],
    )
    def agg_kernel(u_hbm, src_hbm, dst_hbm, z_hbm, out_hbm,
                   acc, sidx_b, didx_b, d0, d1, r0, r1, s0, s1):
        rows_bufs = [r0, r1]
        didx_one = [d0, d1]
        sems = [s0, s1]
        c = lax.axis_index("c")
        s = lax.axis_index("s")

        pltpu.sync_copy(z_hbm.at[pl.ds(s * ZROWS, ZROWS)],
                        acc.at[pl.ds(s * ZROWS, ZROWS)])
        plsc.subcore_barrier()

        cbase = (c * NS + s) * CPT

        @pl.loop(0, CPT // IBLK)
        def _(bk):
            cb = cbase + bk * IBLK
            pltpu.sync_copy(src_hbm.at[pl.ds(cb, IBLK)], sidx_b)
            pltpu.sync_copy(dst_hbm.at[pl.ds(cb, IBLK)], didx_b)

            @pl.loop(0, IBLK // DEPTH)
            def _(k):
                j = k * DEPTH
                copies = []
                for d in range(DEPTH):
                    copies.append(
                        pltpu.async_copy(u_hbm.at[sidx_b.at[j + d]],
                                         rows_bufs[d], sems[d]))
                    for i in range(CHUNK // 16):
                        didx_one[d][pl.ds(i * 16, 16)] = (
                            didx_b[j + d, pl.ds(i * 16, 16)])
                for d in range(DEPTH):
                    copies[d].wait()
                    pltpu.sync_copy(rows_bufs[d], acc.at[didx_one[d]],
                                    add=True)

        plsc.subcore_barrier()
        pltpu.sync_copy(acc.at[pl.ds(s * ZROWS, ZROWS)],
                        out_hbm.at[c].at[pl.ds(s * ZROWS, ZROWS)])

    return agg_kernel(u, src, dst, zeros128)


R = 1000


def _dinv_block(deg_ref):
    d = deg_ref[0, :, 0:1] + deg_ref[1, :, 0:1] + 1.0
    return lax.rsqrt(d)


def _k1_body(x_ref, w_ref, deg_ref, u_ref):
    dinv = _dinv_block(deg_ref)
    h = jnp.dot(x_ref[...], w_ref[...], preferred_element_type=jnp.float32,
                precision=lax.Precision.HIGHEST)
    u_ref[...] = h * dinv


@jax.jit
def _k1(x, w1, deg2):
    return pl.pallas_call(
        _k1_body,
        grid=(N // R,),
        in_specs=[
            pl.BlockSpec((R, F), lambda i: (i, 0)),
            pl.BlockSpec((F, F), lambda i: (0, 0)),
            pl.BlockSpec((NC, R, 16), lambda i: (0, i, 0)),
        ],
        out_specs=pl.BlockSpec((R, F), lambda i: (i, 0)),
        out_shape=jax.ShapeDtypeStruct((N, F), jnp.float32),
    )(x, w1, deg2)


def _k23_body(deg_ref, acc_ref, u_ref, b_ref, w_ref, un_ref):
    dinv = _dinv_block(deg_ref)
    sagg = acc_ref[0] + acc_ref[1] + u_ref[...]
    y = jnp.maximum(sagg * dinv + b_ref[...], 0.0)
    h = jnp.dot(y, w_ref[...], preferred_element_type=jnp.float32,
                precision=lax.Precision.HIGHEST)
    un_ref[...] = h * dinv


@jax.jit
def _k23(deg2, accp, u, b_prev, w_next):
    return pl.pallas_call(
        _k23_body,
        grid=(N // R,),
        in_specs=[
            pl.BlockSpec((NC, R, 16), lambda i: (0, i, 0)),
            pl.BlockSpec((NC, R, F), lambda i: (0, i, 0)),
            pl.BlockSpec((R, F), lambda i: (i, 0)),
            pl.BlockSpec((1, F), lambda i: (0, 0)),
            pl.BlockSpec((F, F), lambda i: (0, 0)),
        ],
        out_specs=pl.BlockSpec((R, F), lambda i: (i, 0)),
        out_shape=jax.ShapeDtypeStruct((N, F), jnp.float32),
    )(deg2, accp, u, b_prev, w_next)


def _k4_body(deg_ref, acc_ref, u_ref, b_ref, out_ref):
    dinv = _dinv_block(deg_ref)
    sagg = acc_ref[0] + acc_ref[1] + u_ref[...]
    out_ref[...] = sagg * dinv + b_ref[...]


@jax.jit
def _k4(deg2, accp, u, b_last):
    return pl.pallas_call(
        _k4_body,
        grid=(N // R,),
        in_specs=[
            pl.BlockSpec((NC, R, 16), lambda i: (0, i, 0)),
            pl.BlockSpec((NC, R, F), lambda i: (0, i, 0)),
            pl.BlockSpec((R, F), lambda i: (i, 0)),
            pl.BlockSpec((1, F), lambda i: (0, 0)),
        ],
        out_specs=pl.BlockSpec((R, F), lambda i: (i, 0)),
        out_shape=jax.ShapeDtypeStruct((N, F), jnp.float32),
    )(deg2, accp, u, b_last)


def kernel(x, edge_index, W1, b1, W2, b2, W3, b3):
    src = edge_index[0].astype(jnp.int32)
    dst = edge_index[1].astype(jnp.int32)
    pad = E_PAD - E
    pad_iota = jnp.arange(pad, dtype=jnp.int32)
    src = jnp.concatenate([src, pad_iota % N]).reshape(NCHUNKS, CHUNK)
    dst = jnp.concatenate([dst, N + pad_iota % (ROWS_PAD - N)]
                          ).reshape(NCHUNKS, CHUNK)

    zeros16 = jnp.zeros((ROWS_PAD, 16), jnp.float32)
    zeros128 = jnp.zeros((ROWS_PAD, F), jnp.float32)

    deg2 = _deg_sc(dst, zeros16)

    u = _k1(x, W1, deg2)
    accp = _agg_sc(u, src, dst, zeros128)
    u = _k23(deg2, accp, u, b1.reshape(1, F), W2)
    accp = _agg_sc(u, src, dst, zeros128)
    u = _k23(deg2, accp, u, b2.reshape(1, F), W3)
    accp = _agg_sc(u, src, dst, zeros128)
    return _k4(deg2, accp, u, b3.reshape(1, F))


# speedup vs baseline: 18.1492x; 1.1061x over previous
import functools
import jax
import jax.numpy as jnp
from jax import lax
from jax.experimental import pallas as pl
from jax.experimental.pallas import tpu as pltpu
from jax.experimental.pallas import tpu_sc as plsc

N = 10000
F = 128
E = 320000
NC = 2
NS = 16
CHUNK = 128
DEPTH = 2
IBLK = 8
DUNROLL = 8
NCHUNKS = NC * NS * IBLK * (-(-E // (CHUNK * NC * NS * IBLK)))
E_PAD = NCHUNKS * CHUNK
CPT = NCHUNKS // (NC * NS)
ROWS_PAD = 10240
ZROWS = ROWS_PAD // NS

_sc_mesh = plsc.VectorSubcoreMesh(core_axis_name="c", subcore_axis_name="s")


@jax.jit
def _deg_sc(dst, zeros16):
    @functools.partial(
        pl.kernel,
        out_type=jax.ShapeDtypeStruct((NC, ROWS_PAD, 16), jnp.float32),
        mesh=_sc_mesh,
        scratch_types=[
            pltpu.VMEM_SHARED((ROWS_PAD, 16), jnp.float32),
            pltpu.VMEM((CPT, CHUNK), jnp.int32),
            [pltpu.VMEM((CHUNK,), jnp.int32) for _ in range(DUNROLL)],
            pltpu.VMEM((CHUNK, 16), jnp.float32),
            pltpu.SemaphoreType.DMA,
        ],
    )
    def deg_kernel(dst_hbm, z_hbm, out_hbm, acc, didx_b, didx_slots,
                   ones_v, sem):
        c = lax.axis_index("c")
        s = lax.axis_index("s")

        @pl.loop(0, CHUNK)
        def _(i):
            ones_v[i, :] = jnp.full((16,), 1.0, jnp.float32)

        cbase = (c * NS + s) * CPT
        pltpu.sync_copy(dst_hbm.at[pl.ds(cbase, CPT)], didx_b)

        pltpu.sync_copy(z_hbm.at[pl.ds(s * ZROWS, ZROWS)],
                        acc.at[pl.ds(s * ZROWS, ZROWS)])
        plsc.subcore_barrier()

        @pl.loop(0, CPT // DUNROLL)
        def _(k):
            copies = []
            for q in range(DUNROLL):
                j = k * DUNROLL + q
                for i in range(CHUNK // 16):
                    didx_slots[q][pl.ds(i * 16, 16)] = (
                        didx_b[j, pl.ds(i * 16, 16)])
                copies.append(pltpu.async_copy(
                    ones_v, acc.at[didx_slots[q]], sem, add=True))
            for cp in copies:
                cp.wait()

        plsc.subcore_barrier()
        pltpu.sync_copy(acc.at[pl.ds(s * ZROWS, ZROWS)],
                        out_hbm.at[c].at[pl.ds(s * ZROWS, ZROWS)])

    return deg_kernel(dst, zeros16)


@jax.jit
def _agg_sc(u, src, dst, zeros128):
    @functools.partial(
        pl.kernel,
        out_type=jax.ShapeDtypeStruct((NC, ROWS_PAD, F), jnp.float32),
        mesh=_sc_mesh,
        scratch_types=[
            pltpu.VMEM_SHARED((ROWS_PAD, F), jnp.float32),
            pltpu.VMEM((IBLK, CHUNK), jnp.int32),
            pltpu.VMEM((IBLK, CHUNK), jnp.int32),
            [pltpu.VMEM((CHUNK,), jnp.int32) for _ in range(DEPTH)],
            [pltpu.VMEM((CHUNK, F), jnp.float32) for _ in range(DEPTH)],
            [pltpu.SemaphoreType.DMA for _ in range(DEPTH)],
            [pltpu.SemaphoreType.DMA for _ in range(DEPTH)],
        ],
    )
    def agg_kernel(u_hbm, src_hbm, dst_hbm, z_hbm, out_hbm,
                   acc, sidx_b, didx_b, didx_one, rows_bufs, gsems, ssems):
        c = lax.axis_index("c")
        s = lax.axis_index("s")

        pltpu.sync_copy(z_hbm.at[pl.ds(s * ZROWS, ZROWS)],
                        acc.at[pl.ds(s * ZROWS, ZROWS)])
        plsc.subcore_barrier()

        cbase = (c * NS + s) * CPT

        @pl.loop(0, CPT // IBLK)
        def _(bk):
            cb = cbase + bk * IBLK
            pltpu.sync_copy(src_hbm.at[pl.ds(cb, IBLK)], sidx_b)
            pltpu.sync_copy(dst_hbm.at[pl.ds(cb, IBLK)], didx_b)

            gath = [None] * DEPTH
            scat = [None] * DEPTH
            for j in range(IBLK + 1):
                if j < IBLK:
                    d = j % DEPTH
                    if scat[d] is not None:
                        scat[d].wait()
                    for i in range(CHUNK // 16):
                        didx_one[d][pl.ds(i * 16, 16)] = (
                            didx_b[j, pl.ds(i * 16, 16)])
                    gath[d] = pltpu.async_copy(u_hbm.at[sidx_b.at[j]],
                                               rows_bufs[d], gsems[d])
                p = j - 1
                if p >= 0:
                    dp = p % DEPTH
                    gath[dp].wait()
                    scat[dp] = pltpu.async_copy(
                        rows_bufs[dp], acc.at[didx_one[dp]], ssems[dp],
                        add=True)
            for d in range(DEPTH):
                if scat[d] is not None:
                    scat[d].wait()

        plsc.subcore_barrier()
        pltpu.sync_copy(acc.at[pl.ds(s * ZROWS, ZROWS)],
                        out_hbm.at[c].at[pl.ds(s * ZROWS, ZROWS)])

    return agg_kernel(u, src, dst, zeros128)


R = 1000


def _dinv_block(deg_ref):
    d = deg_ref[0, :, 0:1] + deg_ref[1, :, 0:1] + 1.0
    return lax.rsqrt(d)


def _k1_body(x_ref, w_ref, deg_ref, u_ref):
    dinv = _dinv_block(deg_ref)
    h = jnp.dot(x_ref[...], w_ref[...], preferred_element_type=jnp.float32,
                precision=lax.Precision.HIGHEST)
    u_ref[...] = h * dinv


@jax.jit
def _k1(x, w1, deg2):
    return pl.pallas_call(
        _k1_body,
        grid=(N // R,),
        in_specs=[
            pl.BlockSpec((R, F), lambda i: (i, 0)),
            pl.BlockSpec((F, F), lambda i: (0, 0)),
            pl.BlockSpec((NC, R, 16), lambda i: (0, i, 0)),
        ],
        out_specs=pl.BlockSpec((R, F), lambda i: (i, 0)),
        out_shape=jax.ShapeDtypeStruct((N, F), jnp.float32),
    )(x, w1, deg2)


def _k23_body(deg_ref, acc_ref, u_ref, b_ref, w_ref, un_ref):
    dinv = _dinv_block(deg_ref)
    sagg = acc_ref[0] + acc_ref[1] + u_ref[...]
    y = jnp.maximum(sagg * dinv + b_ref[...], 0.0)
    h = jnp.dot(y, w_ref[...], preferred_element_type=jnp.float32,
                precision=lax.Precision.HIGHEST)
    un_ref[...] = h * dinv


@jax.jit
def _k23(deg2, accp, u, b_prev, w_next):
    return pl.pallas_call(
        _k23_body,
        grid=(N // R,),
        in_specs=[
            pl.BlockSpec((NC, R, 16), lambda i: (0, i, 0)),
            pl.BlockSpec((NC, R, F), lambda i: (0, i, 0)),
            pl.BlockSpec((R, F), lambda i: (i, 0)),
            pl.BlockSpec((1, F), lambda i: (0, 0)),
            pl.BlockSpec((F, F), lambda i: (0, 0)),
        ],
        out_specs=pl.BlockSpec((R, F), lambda i: (i, 0)),
        out_shape=jax.ShapeDtypeStruct((N, F), jnp.float32),
    )(deg2, accp, u, b_prev, w_next)


def _k4_body(deg_ref, acc_ref, u_ref, b_ref, out_ref):
    dinv = _dinv_block(deg_ref)
    sagg = acc_ref[0] + acc_ref[1] + u_ref[...]
    out_ref[...] = sagg * dinv + b_ref[...]


@jax.jit
def _k4(deg2, accp, u, b_last):
    return pl.pallas_call(
        _k4_body,
        grid=(N // R,),
        in_specs=[
            pl.BlockSpec((NC, R, 16), lambda i: (0, i, 0)),
            pl.BlockSpec((NC, R, F), lambda i: (0, i, 0)),
            pl.BlockSpec((R, F), lambda i: (i, 0)),
            pl.BlockSpec((1, F), lambda i: (0, 0)),
        ],
        out_specs=pl.BlockSpec((R, F), lambda i: (i, 0)),
        out_shape=jax.ShapeDtypeStruct((N, F), jnp.float32),
    )(deg2, accp, u, b_last)


def kernel(x, edge_index, W1, b1, W2, b2, W3, b3):
    src = edge_index[0].astype(jnp.int32)
    dst = edge_index[1].astype(jnp.int32)
    pad = E_PAD - E
    pad_iota = jnp.arange(pad, dtype=jnp.int32)
    src = jnp.concatenate([src, pad_iota % N]).reshape(NCHUNKS, CHUNK)
    dst = jnp.concatenate([dst, N + pad_iota % (ROWS_PAD - N)]
                          ).reshape(NCHUNKS, CHUNK)

    zeros16 = jnp.zeros((ROWS_PAD, 16), jnp.float32)
    zeros128 = jnp.zeros((ROWS_PAD, F), jnp.float32)

    deg2 = _deg_sc(dst, zeros16)

    u = _k1(x, W1, deg2)
    accp = _agg_sc(u, src, dst, zeros128)
    u = _k23(deg2, accp, u, b1.reshape(1, F), W2)
    accp = _agg_sc(u, src, dst, zeros128)
    u = _k23(deg2, accp, u, b2.reshape(1, F), W3)
    accp = _agg_sc(u, src, dst, zeros128)
    return _k4(deg2, accp, u, b3.reshape(1, F))
